# Initial kernel scaffold; baseline (speedup 1.0000x reference)
import jax
import jax.numpy as jnp
from jax.experimental import pallas as pl


def kernel(var_idx, const_idx, edge_vals, const_values, objective_multipliers, integer_mask, Wc1, bc1, Wc2, bc2, Wq1, bq1, Wq2, bq2, Wv1, bv1, Wv2, bv2, Wo1, bo1, Wo2, bo2):
    raise NotImplementedError("write your pallas kernel here")


# speedup vs baseline: 3.9027x; 3.9027x over previous
import functools

import jax
import jax.numpy as jnp
from jax import lax
from jax.experimental import pallas as pl
from jax.experimental.pallas import tpu as pltpu
from jax.experimental.pallas import tpu_sc as plsc

V, C, E, F, OB = 50000, 12500, 800000, 64, 1
UPDATE_STEPS, PASSING_STEPS = 3, 2

NC, NS, L = 2, 16, 16
VHALF = 25088
Vp = 2 * VHALF
Cp = 12800
Ep = 802816
EPW = Ep // 32
ERW = EPW // 128
CH = 512
CHR = CH // 128
ACC3 = 25600

_mesh = plsc.VectorSubcoreMesh(core_axis_name="c", subcore_axis_name="s",
                               num_cores=NC, num_subcores=NS)
_sc_params = pltpu.CompilerParams(use_tc_tiling_on_sc=False)


def _zero_zbuf(zbuf, rows):
    z = jnp.zeros((L,), jnp.float32)

    @pl.loop(0, rows)
    def _(r):
        for k in range(4):
            zbuf[r, pl.ds(k * L, L)] = z


def _scale_rows(rows, ev_v):
    @pl.loop(0, CH // L)
    def _(g):
        ev16 = ev_v[pl.ds(g * L, L)]
        for i in range(L):
            e = g * L + i
            s = ev16[i]
            for k in range(4):
                sl = (e, pl.ds(k * L, L))
                rows[sl] = rows[sl] * s


@functools.partial(
    pl.kernel,
    out_type=(jax.ShapeDtypeStruct((Vp,), jnp.float32),
              jax.ShapeDtypeStruct((Vp,), jnp.float32),
              jax.ShapeDtypeStruct((Cp,), jnp.float32),
              jax.ShapeDtypeStruct((Cp,), jnp.float32)),
    mesh=_mesh,
    scratch_types=(
        pltpu.VMEM((CHR, 128), jnp.int32),
        pltpu.VMEM((CHR, 128), jnp.int32),
        pltpu.VMEM((CH,), jnp.float32),
        pltpu.VMEM((3136,), jnp.float32),
        pltpu.VMEM_SHARED((Vp,), jnp.float32),
        pltpu.VMEM_SHARED((Cp,), jnp.float32),
        pltpu.SemaphoreType.DMA,
    ),
    compiler_params=_sc_params,
)
def _sc_scalers(vi_hbm, ci_hbm, ev_hbm, out_sv0, out_sv1, out_sc0, out_sc1,
                vidx, cidx, ev_v, zbuf, acc_v, acc_c, sem):
    core = lax.axis_index("c")
    sub = lax.axis_index("s")
    w = core * NS + sub

    z = jnp.zeros((L,), jnp.float32)

    @pl.loop(0, 3136 // L)
    def _(r):
        zbuf[pl.ds(r * L, L)] = z

    pltpu.sync_copy(zbuf, acc_v.at[pl.ds(sub * 3136, 3136)])
    pltpu.sync_copy(zbuf.at[pl.ds(0, 800)], acc_c.at[pl.ds(sub * 800, 800)])
    plsc.subcore_barrier()

    @pl.loop(0, EPW // CH)
    def _(c):
        r0 = w * ERW + c * CHR
        e0 = w * EPW + c * CH
        pltpu.sync_copy(vi_hbm.at[pl.ds(r0, CHR)], vidx)
        pltpu.sync_copy(ci_hbm.at[pl.ds(r0, CHR)], cidx)
        pltpu.sync_copy(ev_hbm.at[pl.ds(e0, CH)], ev_v)
        for j in range(CHR):
            pltpu.sync_copy(ev_v.at[pl.ds(j * 128, 128)],
                            acc_v.at[vidx.at[j]], add=True)
            pltpu.sync_copy(ev_v.at[pl.ds(j * 128, 128)],
                            acc_c.at[cidx.at[j]], add=True)

    plsc.subcore_barrier()

    pltpu.sync_copy(acc_v.at[pl.ds(sub * 3136, 3136)], zbuf)

    @pl.when(core == 0)
    def _():
        pltpu.sync_copy(zbuf, out_sv0.at[pl.ds(sub * 3136, 3136)])

    @pl.when(core == 1)
    def _():
        pltpu.sync_copy(zbuf, out_sv1.at[pl.ds(sub * 3136, 3136)])

    pltpu.sync_copy(acc_c.at[pl.ds(sub * 800, 800)], zbuf.at[pl.ds(0, 800)])

    @pl.when(core == 0)
    def _():
        pltpu.sync_copy(zbuf.at[pl.ds(0, 800)], out_sc0.at[pl.ds(sub * 800, 800)])

    @pl.when(core == 1)
    def _():
        pltpu.sync_copy(zbuf.at[pl.ds(0, 800)], out_sc1.at[pl.ds(sub * 800, 800)])


@functools.partial(
    pl.kernel,
    out_type=jax.ShapeDtypeStruct((NC, Cp, F), jnp.float32),
    mesh=_mesh,
    scratch_types=(
        pltpu.VMEM((CHR, 128), jnp.int32),
        pltpu.VMEM((CHR, 128), jnp.int32),
        pltpu.VMEM((CH,), jnp.float32),
        pltpu.VMEM((CH, F), jnp.float32),
        pltpu.VMEM((160, F), jnp.float32),
        pltpu.VMEM_SHARED((Cp, F), jnp.float32),
        pltpu.SemaphoreType.DMA,
    ),
    compiler_params=_sc_params,
)
def _sc_spmm_t(vi_hbm, ci_hbm, ev_hbm, x_hbm, out_hbm,
               vidx, cidx, ev_v, rows, zbuf, acc, sem):
    core = lax.axis_index("c")
    sub = lax.axis_index("s")
    w = core * NS + sub

    _zero_zbuf(zbuf, 160)

    @pl.loop(0, 5)
    def _(i):
        pltpu.sync_copy(zbuf, acc.at[pl.ds(sub * 800 + i * 160, 160)])

    plsc.subcore_barrier()

    @pl.loop(0, EPW // CH)
    def _(c):
        r0 = w * ERW + c * CHR
        e0 = w * EPW + c * CH
        pltpu.sync_copy(vi_hbm.at[pl.ds(r0, CHR)], vidx)
        pltpu.sync_copy(ci_hbm.at[pl.ds(r0, CHR)], cidx)
        pltpu.sync_copy(ev_hbm.at[pl.ds(e0, CH)], ev_v)
        cps = [pltpu.async_copy(x_hbm.at[vidx.at[j]],
                                rows.at[pl.ds(j * 128, 128)], sem)
               for j in range(CHR)]
        for cp in cps:
            cp.wait()
        _scale_rows(rows, ev_v)
        for j in range(CHR):
            pltpu.sync_copy(rows.at[pl.ds(j * 128, 128)],
                            acc.at[cidx.at[j]], add=True)

    plsc.subcore_barrier()

    @pl.loop(0, 5)
    def _(i):
        pltpu.sync_copy(acc.at[pl.ds(sub * 800 + i * 160, 160)], zbuf)
        pltpu.sync_copy(zbuf, out_hbm.at[core, pl.ds(sub * 800 + i * 160, 160)])


FH = F // 2


@functools.partial(
    pl.kernel,
    out_type=jax.ShapeDtypeStruct((NC, Vp, FH), jnp.float32),
    mesh=_mesh,
    scratch_types=(
        pltpu.VMEM((CHR, 128), jnp.int32),
        pltpu.VMEM((CHR, 128), jnp.int32),
        pltpu.VMEM((CHR, 128), jnp.int32),
        pltpu.VMEM((CH,), jnp.float32),
        pltpu.VMEM((CH, FH), jnp.float32),
        pltpu.VMEM((160, FH), jnp.float32),
        pltpu.VMEM_SHARED((ACC3, FH), jnp.float32),
        pltpu.SemaphoreType.DMA,
    ),
    compiler_params=_sc_params,
)
def _sc_spmm(vi_hbm, ci_hbm, ev_hbm, y2_hbm, out_hbm,
             cidx, vidx, sidx, ev_v, rows, zbuf, acc, sem):
    core = lax.axis_index("c")
    sub = lax.axis_index("s")
    ept = Ep // NS
    ert = ept // 128

    z = jnp.zeros((L,), jnp.float32)

    @pl.loop(0, 160)
    def _(r):
        for k in range(FH // L):
            zbuf[r, pl.ds(k * L, L)] = z

    lanes = lax.iota(jnp.int32, L)

    for p in range(2):
        vbase = p * VHALF

        @pl.loop(0, ACC3 // NS // 160)
        def _(i):
            pltpu.sync_copy(zbuf,
                            acc.at[pl.ds(sub * (ACC3 // NS) + i * 160, 160)])

        plsc.subcore_barrier()

        @pl.loop(0, ept // CH)
        def _(c):
            r0 = sub * ert + c * CHR
            e0 = sub * ept + c * CH
            pltpu.sync_copy(ci_hbm.at[pl.ds(r0, CHR)], cidx)
            pltpu.sync_copy(vi_hbm.at[pl.ds(r0, CHR)], vidx)
            pltpu.sync_copy(ev_hbm.at[pl.ds(e0, CH)], ev_v)
            cps = [pltpu.async_copy(y2_hbm.at[core].at[cidx.at[j]],
                                    rows.at[pl.ds(j * 128, 128)], sem)
                   for j in range(CHR)]
            for cp in cps:
                cp.wait()
            for j in range(CHR):
                for k in range(128 // L):
                    loc = vidx[j, pl.ds(k * L, L)] - vbase
                    ok = (loc >= 0) & (loc < VHALF)
                    dump = VHALF + lanes * 32 + (j * 8 + k)
                    sidx[j, pl.ds(k * L, L)] = jnp.where(ok, loc, dump)

            @pl.loop(0, CH // L)
            def _(g):
                ev16 = ev_v[pl.ds(g * L, L)]
                for i in range(L):
                    e = g * L + i
                    s = ev16[i]
                    for k in range(FH // L):
                        sl = (e, pl.ds(k * L, L))
                        rows[sl] = rows[sl] * s

            for j in range(CHR):
                pltpu.sync_copy(rows.at[pl.ds(j * 128, 128)],
                                acc.at[sidx.at[j]], add=True)

        plsc.subcore_barrier()

        @pl.loop(0, 4)
        def _(i):
            pltpu.sync_copy(acc.at[pl.ds(sub * 1568 + i * 392, 392)],
                            rows.at[pl.ds(0, 392)])
            pltpu.sync_copy(
                rows.at[pl.ds(0, 392)],
                out_hbm.at[core, pl.ds(vbase + sub * 1568 + i * 392, 392)])

        plsc.subcore_barrier()


def _nodenorm(x):
    m = jnp.mean(x, axis=-1, keepdims=True)
    d = x - m
    s = jnp.sqrt(jnp.mean(d * d, axis=-1, keepdims=True))
    return d / (s + 1e-5)


def _dot(a, b):
    return jax.lax.dot_general(a, b, (((1,), (0,)), ((), ())),
                               preferred_element_type=jnp.float32)


def _recip2_body(s0, s1, o):
    o[...] = 1.0 / jnp.maximum(s0[...] + s1[...], 1e-9)


def _recip2(s0, s1):
    r = s0.shape[0] // 128
    s0 = s0.reshape(r, 128)
    s1 = s1.reshape(r, 128)
    out = pl.pallas_call(
        _recip2_body,
        out_shape=jax.ShapeDtypeStruct((r, 128), jnp.float32),
    )(s0, s1)
    return out.reshape(r * 128, 1)


def _axpy2_body(x2, p0, p1, rcp, o2):
    x = jnp.concatenate([x2[0], x2[1]], axis=-1)
    r = x + (p0[...] + p1[...]) * rcp[...]
    o2[0] = r[:, :FH]
    o2[1] = r[:, FH:]


def _axpy2(x2, p0, p1, rcp, blk):
    n = p0.shape[0]
    grid = n // blk
    bs2 = pl.BlockSpec((2, blk, FH), lambda i: (0, i, 0))
    bs = pl.BlockSpec((blk, F), lambda i: (i, 0))
    bs1 = pl.BlockSpec((blk, 1), lambda i: (i, 0))
    return pl.pallas_call(
        _axpy2_body,
        grid=(grid,),
        in_specs=[bs2, bs, bs, bs1],
        out_specs=bs2,
        out_shape=jax.ShapeDtypeStruct((2, n, FH), jnp.float32),
    )(x2, p0, p1, rcp)


def _axpy1_body(x, p2, rcp, o):
    p = jnp.concatenate([p2[0], p2[1]], axis=-1)
    o[...] = x[...] + p * rcp[...]


def _axpy1(x, p2, rcp, blk):
    n = x.shape[0]
    grid = n // blk
    bs = pl.BlockSpec((blk, F), lambda i: (i, 0))
    bs2 = pl.BlockSpec((2, blk, FH), lambda i: (0, i, 0))
    bs1 = pl.BlockSpec((blk, 1), lambda i: (i, 0))
    return pl.pallas_call(
        _axpy1_body,
        grid=(grid,),
        in_specs=[bs, bs2, bs1],
        out_specs=bs,
        out_shape=jax.ShapeDtypeStruct(x.shape, jnp.float32),
    )(x, p2, rcp)


def _w_spec(shape):
    return pl.BlockSpec(shape, lambda i: tuple(0 for _ in shape))


def _qmlp_body(x1, x2, wa, wb, b1, w2, b2, o):
    q = _dot(x1[...], wa[...]) + _dot(x2[...], wb[...]) + b1[...]
    q = _dot(jnp.maximum(q, 0.0), w2[...]) + b2[...]
    o[...] = jax.nn.sigmoid(q)


def _qmlp(x1, x2, wa, wb, b1, w2, b2, blk):
    grid = x1.shape[0] // blk
    bs = pl.BlockSpec((blk, F), lambda i: (i, 0))
    return pl.pallas_call(
        _qmlp_body,
        grid=(grid,),
        in_specs=[bs, bs, _w_spec((F, F)), _w_spec((F, F)),
                  _w_spec((1, F)), _w_spec((F, F)), _w_spec((1, F))],
        out_specs=bs,
        out_shape=jax.ShapeDtypeStruct(x1.shape, jnp.float32),
    )(x1, x2, wa, wb, b1, w2, b2)


def _hmlp_body(cst2, tc2, l0, l1, cv, rcp, wa, wb, wc, wd, b1, w2, b2, o2):
    cst = jnp.concatenate([cst2[0], cst2[1]], axis=-1)
    tc = jnp.concatenate([tc2[0], tc2[1]], axis=-1)
    left = l0[...] + l1[...]
    cvb = cv[...]
    rc = rcp[...]
    closs = jnp.maximum(left - cvb, 0.0) * rc
    closs1 = jnp.maximum(cvb - left, 0.0) * rc
    h = (_dot(cst, wa[...]) + _dot(tc, wb[...]) +
         _dot(closs, wc[...]) + _dot(closs1, wd[...]) + b1[...])
    h = jnp.maximum(_nodenorm(h), 0.0)
    r = _dot(h, w2[...]) + b2[...]
    o2[0] = r[:, :FH]
    o2[1] = r[:, FH:]


def _hmlp(cst2, tc2, l0, l1, cv, rcp, wa, wb, wc, wd, b1, w2, b2, blk):
    n = l0.shape[0]
    grid = n // blk
    bs2 = pl.BlockSpec((2, blk, FH), lambda i: (0, i, 0))
    bs = pl.BlockSpec((blk, F), lambda i: (i, 0))
    bs1 = pl.BlockSpec((blk, 1), lambda i: (i, 0))
    return pl.pallas_call(
        _hmlp_body,
        grid=(grid,),
        in_specs=[bs2, bs2, bs, bs, bs1, bs1,
                  _w_spec((F, F)), _w_spec((F, F)), _w_spec((F, F)),
                  _w_spec((F, F)), _w_spec((1, F)), _w_spec((F, F)),
                  _w_spec((1, F))],
        out_specs=bs2,
        out_shape=jax.ShapeDtypeStruct((2, n, FH), jnp.float32),
    )(cst2, tc2, l0, l1, cv, rcp, wa, wb, wc, wd, b1, w2, b2)


def _gomlp_body(var, tv, qry, obj, wa, wb, wc, wrow, b1, w2, b2,
                wo1, bo1, wo2, bo2, vo, oo, so):
    ob = obj[...]
    g = (_dot(var[...], wa[...]) + _dot(tv[...], wb[...]) +
         _dot(qry[...] * ob, wc[...]) + ob * wrow[...] + b1[...])
    g = jnp.maximum(_nodenorm(g), 0.0)
    vnew = _dot(g, w2[...]) + b2[...]
    vo[...] = vnew
    o = _dot(vnew, wo1[...]) + bo1[...]
    o = jnp.maximum(_nodenorm(o), 0.0)
    o2 = _dot(o, wo2[...]) + bo2[...]
    oo[...] = o2
    so[...] = jax.nn.sigmoid(o2)


def _gomlp(var, tv, qry, obj, wa, wb, wc, wrow, b1, w2, b2,
           wo1, bo1, wo2, bo2, blk):
    n = var.shape[0]
    grid = n // blk
    bs = pl.BlockSpec((blk, F), lambda i: (i, 0))
    bs1 = pl.BlockSpec((blk, 1), lambda i: (i, 0))
    return pl.pallas_call(
        _gomlp_body,
        grid=(grid,),
        in_specs=[bs, bs, bs, bs1,
                  _w_spec((F, F)), _w_spec((F, F)), _w_spec((F, F)),
                  _w_spec((1, F)), _w_spec((1, F)), _w_spec((F, F)),
                  _w_spec((1, F)), _w_spec((F, F)), _w_spec((1, F)),
                  _w_spec((F, OB)), _w_spec((1, OB))],
        out_specs=(bs, bs1, bs1),
        out_shape=(jax.ShapeDtypeStruct((n, F), jnp.float32),
                   jax.ShapeDtypeStruct((n, OB), jnp.float32),
                   jax.ShapeDtypeStruct((n, OB), jnp.float32)),
    )(var, tv, qry, obj, wa, wb, wc, wrow, b1, w2, b2, wo1, bo1, wo2, bo2)


def kernel(var_idx, const_idx, edge_vals, const_values, objective_multipliers,
           integer_mask, Wc1, bc1, Wc2, bc2, Wq1, bq1, Wq2, bq2,
           Wv1, bv1, Wv2, bv2, Wo1, bo1, Wo2, bo2):
    pad_e = Ep - E
    vi = jnp.concatenate([var_idx.astype(jnp.int32),
                          jnp.zeros((pad_e,), jnp.int32)]).reshape(Ep // 128, 128)
    ci = jnp.concatenate([const_idx.astype(jnp.int32),
                          jnp.zeros((pad_e,), jnp.int32)]).reshape(Ep // 128, 128)
    ev = jnp.concatenate([edge_vals, jnp.zeros((pad_e,), jnp.float32)])

    cv = jnp.concatenate([const_values,
                          jnp.zeros((Cp - C,), jnp.float32)]).reshape(Cp, 1)
    obj = jnp.concatenate([objective_multipliers,
                           jnp.zeros((Vp - V,), jnp.float32)]).reshape(Vp, 1)

    sv0, sv1, sc0, sc1 = _sc_scalers(vi, ci, ev)
    rcp_v = _recip2(sv0, sv1)
    rcp_c = _recip2(sc0, sc1)

    b1r = lambda b: b.reshape(1, -1)
    Wq1a, Wq1b = Wq1[:F], Wq1[F:]
    Wc1a, Wc1b, Wc1c, Wc1d = Wc1[:F], Wc1[F:2 * F], Wc1[2 * F:3 * F], Wc1[3 * F:]
    Wv1a, Wv1b, Wv1c = Wv1[:F], Wv1[F:2 * F], Wv1[2 * F:3 * F]
    Wv1row = Wv1[3 * F].reshape(1, F)

    variables = jnp.ones((Vp, F), jnp.float32)
    constraints2 = jnp.ones((2, Cp, FH), jnp.float32)

    VB, CB = 1024, 1600
    sigs = []
    out_vars = None
    for _ in range(UPDATE_STEPS):
        tv = variables
        tc2 = constraints2
        for _ in range(PASSING_STEPS):
            pt = _sc_spmm_t(vi, ci, ev, tv)
            tc2 = _axpy2(tc2, pt[0], pt[1], rcp_c, CB)
            pv2 = _sc_spmm(vi, ci, ev, tc2)
            tv = _axpy1(tv, pv2, rcp_v, VB)
        query = _qmlp(variables, tv, Wq1a, Wq1b, b1r(bq1), Wq2, b1r(bq2), VB)
        left = _sc_spmm_t(vi, ci, ev, query)
        constraints2 = _hmlp(constraints2, tc2, left[0], left[1], cv, rcp_c,
                             Wc1a, Wc1b, Wc1c, Wc1d, b1r(bc1), Wc2, b1r(bc2),
                             CB)
        variables, out_vars, sg = _gomlp(
            variables, tv, query, obj, Wv1a, Wv1b, Wv1c, Wv1row, b1r(bv1),
            Wv2, b1r(bv2), Wo1, b1r(bo1), Wo2, b1r(bo2), VB)
        sigs.append(sg)

    out_stack = jnp.stack([s[:V] for s in sigs], axis=0)
    return out_stack, out_vars[:V]
